# baseline (speedup 1.0000x reference)
import functools

import jax
import jax.numpy as jnp
from jax import lax
from jax.experimental import pallas as pl
from jax.experimental.pallas import tpu as pltpu
from jax.experimental.pallas import tpu_sc as plsc

N = 10000
M = 10240
E = 320000
NC, NS, NW = 2, 16, 32
K = 128
EPW = 10496
NCHUNK = EPW // K
EP = NW * EPW
RPT = 624
F32 = jnp.float32
I32 = jnp.int32

_MESH = dict(core_axis_name="c", subcore_axis_name="s",
             num_cores=NC, num_subcores=NS)


def _sc_degree(dst_w, w_w):
    @functools.partial(
        pl.kernel,
        out_type=jax.ShapeDtypeStruct((NW, M), F32),
        mesh=plsc.VectorSubcoreMesh(**_MESH),
        compiler_params=pltpu.CompilerParams(needs_layout_passes=False),
        scratch_types=[
            pltpu.VMEM((NCHUNK, K), I32),
            pltpu.VMEM((NCHUNK, K), F32),
            pltpu.VMEM((M,), F32),
        ],
    )
    def deg_kernel(dst_hbm, w_hbm, degp_hbm, dst_v, w_v, acc_t):
        wid = lax.axis_index("c") * NS + lax.axis_index("s")
        pltpu.sync_copy(dst_hbm.at[wid], dst_v)
        pltpu.sync_copy(w_hbm.at[wid], w_v)

        def zbody(j, _):
            acc_t[pl.ds(j * 16, 16)] = jnp.zeros((16,), F32)
            return 0
        lax.fori_loop(0, M // 16, zbody, 0)

        def kbody(k, _):
            for j in range(K // 16):
                sl = pl.ds(j * 16, 16)
                plsc.addupdate_scatter(acc_t, [dst_v[k, sl]], w_v[k, sl])
            return 0
        lax.fori_loop(0, NCHUNK, kbody, 0)
        pltpu.sync_copy(acc_t, degp_hbm.at[wid])

    return deg_kernel(dst_w, w_w)


def _sc_spmm(table, src_w, dst_w, w_w, zeros2d, C):
    @functools.partial(
        pl.kernel,
        out_type=jax.ShapeDtypeStruct((NC, C, M, 128), F32),
        mesh=plsc.VectorSubcoreMesh(**_MESH),
        compiler_params=pltpu.CompilerParams(needs_layout_passes=False),
        scratch_types=[
            pltpu.VMEM((K,), I32), pltpu.VMEM((K,), I32),
            pltpu.VMEM((K,), I32), pltpu.VMEM((K,), I32),
            pltpu.VMEM((K,), F32), pltpu.VMEM((K,), F32),
            pltpu.VMEM((K,), I32), pltpu.VMEM((K,), I32),
            pltpu.VMEM((K,), I32), pltpu.VMEM((K,), I32),
            pltpu.VMEM((K, 128), F32), pltpu.VMEM((K, 128), F32),
            pltpu.VMEM_SHARED((N, 128), F32),
            pltpu.SemaphoreType.DMA, pltpu.SemaphoreType.DMA,
            pltpu.SemaphoreType.DMA, pltpu.SemaphoreType.DMA,
            pltpu.SemaphoreType.DMA, pltpu.SemaphoreType.DMA,
        ],
    )
    def spmm_kernel(table_hbm, src_hbm, dst_hbm, w_hbm, z_hbm, out_hbm,
                    src0, src1, dst0, dst1, w0, w1, idx0, idx1,
                    sdst0, sdst1, rows0, rows1, acc,
                    sg0, sg1, se0, se1, ss0, ss1):
        cid = lax.axis_index("c")
        sid = lax.axis_index("s")
        wid = cid * NS + sid
        rsl = pl.ds(sid * RPT, RPT)
        tail = pl.ds(NS * RPT, N - NS * RPT)
        SRC, DST, W = (src0, src1), (dst0, dst1), (w0, w1)
        IDX, ROWS = (idx0, idx1), (rows0, rows1)
        SDST = (sdst0, sdst1)
        SG, SE, SS = (sg0, sg1), (se0, se1), (ss0, ss1)
        NH = NCHUNK // 2

        def fetch_start(k, p):
            pltpu.async_copy(src_hbm.at[wid, k], SRC[p], SE[p])
            pltpu.async_copy(dst_hbm.at[wid, k], DST[p], SE[p])
            pltpu.async_copy(w_hbm.at[wid, k], W[p], SE[p])

        def fetch_wait(k, p):
            pltpu.make_async_copy(src_hbm.at[wid, k], SRC[p], SE[p]).wait()
            pltpu.make_async_copy(dst_hbm.at[wid, k], DST[p], SE[p]).wait()
            pltpu.make_async_copy(w_hbm.at[wid, k], W[p], SE[p]).wait()

        def idx_compute(p, c):
            for j in range(K // 16):
                sl = pl.ds(j * 16, 16)
                IDX[p][sl] = SRC[p][sl] + (c * M)
                SDST[p][sl] = DST[p][sl]

        def scale(p):
            def ebody(e, _):
                e16 = jnp.full((16,), e, I32)
                wv = plsc.load_gather(W[p], [e16])
                for j in range(8):
                    sl = pl.ds(j * 16, 16)
                    ROWS[p][e, sl] = ROWS[p][e, sl] * wv
                return 0
            lax.fori_loop(0, K, ebody, 0, unroll=8)

        for c in range(C):
            pltpu.sync_copy(z_hbm.at[rsl], acc.at[rsl])

            @pl.when(sid == 0)
            def _():
                pltpu.sync_copy(z_hbm.at[tail], acc.at[tail])

            plsc.subcore_barrier()

            pltpu.sync_copy(src_hbm.at[wid, 0], SRC[0])
            pltpu.sync_copy(dst_hbm.at[wid, 0], DST[0])
            pltpu.sync_copy(w_hbm.at[wid, 0], W[0])
            idx_compute(0, c)
            pltpu.async_copy(table_hbm.at[IDX[0]], ROWS[0], SG[0])
            fetch_start(1, 1)

            def kbody(i, _, c=c):
                for p in (0, 1):
                    q = 1 - p
                    k = 2 * i + p
                    pltpu.make_async_copy(
                        table_hbm.at[IDX[p]], ROWS[p], SG[p]).wait()

                    def prev_scatter_wait():
                        pltpu.make_async_copy(
                            ROWS[q], acc.at[SDST[q]], SS[q]).wait()

                    def next_gather():
                        fetch_wait(k + 1, q)
                        idx_compute(q, c)
                        pltpu.async_copy(table_hbm.at[IDX[q]], ROWS[q], SG[q])

                    if p == 0:
                        @pl.when(i > 0)
                        def _():
                            prev_scatter_wait()
                        next_gather()
                    else:
                        prev_scatter_wait()

                        @pl.when(i < NH - 1)
                        def _():
                            next_gather()

                    scale(p)
                    pltpu.async_copy(ROWS[p], acc.at[SDST[p]], SS[p],
                                     add=True)

                    @pl.when(i < NH - 1)
                    def _():
                        fetch_start(k + 2, p)
                return 0

            lax.fori_loop(0, NH, kbody, 0)
            pltpu.make_async_copy(ROWS[1], acc.at[SDST[1]], SS[1]).wait()
            plsc.subcore_barrier()
            pltpu.sync_copy(acc.at[rsl], out_hbm.at[cid, c, rsl])

            @pl.when(sid == 0)
            def _():
                pltpu.sync_copy(acc.at[tail], out_hbm.at[cid, c, tail])

            plsc.subcore_barrier()

    return spmm_kernel(table, src_w, dst_w, w_w, zeros2d)


def _tc_dis_scale(degp, x_p):
    BM = 2048

    def body(degp_ref, x_ref, dis_ref, xs_ref):
        deg = jnp.sum(degp_ref[...], axis=0)
        dis = jnp.where(deg > 0, lax.rsqrt(deg), 0.0)[:, None]
        dis_ref[...] = dis
        xs_ref[...] = x_ref[...] * dis

    return pl.pallas_call(
        body,
        grid=(M // BM,),
        in_specs=[
            pl.BlockSpec((NW, BM), lambda i: (0, i)),
            pl.BlockSpec((BM, 128), lambda i: (i, 0)),
        ],
        out_specs=[
            pl.BlockSpec((BM, 1), lambda i: (i, 0)),
            pl.BlockSpec((BM, 128), lambda i: (i, 0)),
        ],
        out_shape=[
            jax.ShapeDtypeStruct((M, 1), F32),
            jax.ShapeDtypeStruct((M, 128), F32),
        ],
    )(degp, x_p)


def _tc_mm1_stats(acc0, dis, W1, b1):
    BM = 1024

    def body(acc_ref, dis_ref, w_ref, b_ref, y_ref, st_ref):
        i = pl.program_id(0)
        lhs = (acc_ref[0] + acc_ref[1]) * dis_ref[...]
        y = jnp.dot(lhs, w_ref[...], preferred_element_type=F32) + b_ref[...]
        y_ref[...] = y
        rows = i * BM + lax.broadcasted_iota(I32, (BM, 1), 0)
        ym = jnp.where(rows < N, y, 0.0)
        st = jnp.concatenate([jnp.sum(ym, axis=0, keepdims=True),
                              jnp.sum(ym * ym, axis=0, keepdims=True)], axis=0)

        @pl.when(i == 0)
        def _():
            st_ref[...] = st

        @pl.when(i > 0)
        def _():
            st_ref[...] = st_ref[...] + st

    return pl.pallas_call(
        body,
        grid=(M // BM,),
        in_specs=[
            pl.BlockSpec((2, BM, 128), lambda i: (0, i, 0)),
            pl.BlockSpec((BM, 1), lambda i: (i, 0)),
            pl.BlockSpec((128, 1024), lambda i: (0, 0)),
            pl.BlockSpec((1, 1024), lambda i: (0, 0)),
        ],
        out_specs=[
            pl.BlockSpec((BM, 1024), lambda i: (i, 0)),
            pl.BlockSpec((2, 1024), lambda i: (0, 0)),
        ],
        out_shape=[
            jax.ShapeDtypeStruct((M, 1024), F32),
            jax.ShapeDtypeStruct((2, 1024), F32),
        ],
    )(acc0, dis, W1, b1)


def _tc_bn_mm2(Y1, st1, g1, be1, W2, dis):
    BM = 512

    def body(y_ref, st_ref, g_ref, be_ref, w_ref, dis_ref, out_ref):
        m = st_ref[0] * (1.0 / N)
        v = st_ref[1] * (1.0 / N) - m * m
        scale = lax.rsqrt(v + 1e-5) * g_ref[0]
        shift = be_ref[0] - m * scale
        f = jnp.maximum(y_ref[...] * scale[None, :] + shift[None, :], 0.0)
        z = jnp.dot(f, w_ref[...], preferred_element_type=F32) * dis_ref[...]
        for c in range(8):
            out_ref[c] = z[:, c * 128:(c + 1) * 128]

    return pl.pallas_call(
        body,
        grid=(M // BM,),
        in_specs=[
            pl.BlockSpec((BM, 1024), lambda i: (i, 0)),
            pl.BlockSpec((2, 1024), lambda i: (0, 0)),
            pl.BlockSpec((1, 1024), lambda i: (0, 0)),
            pl.BlockSpec((1, 1024), lambda i: (0, 0)),
            pl.BlockSpec((1024, 1024), lambda i: (0, 0)),
            pl.BlockSpec((BM, 1), lambda i: (i, 0)),
        ],
        out_specs=pl.BlockSpec((8, BM, 128), lambda i: (0, i, 0)),
        out_shape=jax.ShapeDtypeStruct((8, M, 128), F32),
    )(Y1, st1, g1, be1, W2, dis)


def _tc_stats2(acc2, dis, b2):
    BM = 512

    def body(acc_ref, dis_ref, b_ref, st_ref):
        i = pl.program_id(0)
        rows = i * BM + lax.broadcasted_iota(I32, (BM, 1), 0)
        mask = rows < N
        ss_list, s_list = [], []
        for c in range(8):
            y = (acc_ref[0, c] + acc_ref[1, c]) * dis_ref[...] + b_ref[c]
            ym = jnp.where(mask, y, 0.0)
            s_list.append(jnp.sum(ym, axis=0))
            ss_list.append(jnp.sum(ym * ym, axis=0))
        st = jnp.stack([jnp.stack(s_list), jnp.stack(ss_list)])

        @pl.when(i == 0)
        def _():
            st_ref[...] = st

        @pl.when(i > 0)
        def _():
            st_ref[...] = st_ref[...] + st

    return pl.pallas_call(
        body,
        grid=(M // BM,),
        in_specs=[
            pl.BlockSpec((2, 8, BM, 128), lambda i: (0, 0, i, 0)),
            pl.BlockSpec((BM, 1), lambda i: (i, 0)),
            pl.BlockSpec((8, 128), lambda i: (0, 0)),
        ],
        out_specs=pl.BlockSpec((2, 8, 128), lambda i: (0, 0, 0)),
        out_shape=jax.ShapeDtypeStruct((2, 8, 128), F32),
    )(acc2, dis, b2)


def _tc_bn_mm3(acc2, dis, b2, st2, g2, be2, W3):
    BM = 512

    def body(acc_ref, dis_ref, b_ref, st_ref, g_ref, be_ref, w_ref, out_ref):
        d = dis_ref[...]
        fs = []
        for c in range(8):
            m = st_ref[0, c] * (1.0 / N)
            v = st_ref[1, c] * (1.0 / N) - m * m
            scale = lax.rsqrt(v + 1e-5) * g_ref[c]
            shift = be_ref[c] - m * scale
            y = (acc_ref[0, c] + acc_ref[1, c]) * d + b_ref[c]
            fs.append(jnp.maximum(y * scale[None, :] + shift[None, :], 0.0))
        f = jnp.concatenate(fs, axis=1)
        z = jnp.dot(f, w_ref[...], preferred_element_type=F32) * d
        for c in range(4):
            out_ref[c] = z[:, c * 128:(c + 1) * 128]

    return pl.pallas_call(
        body,
        grid=(M // BM,),
        in_specs=[
            pl.BlockSpec((2, 8, BM, 128), lambda i: (0, 0, i, 0)),
            pl.BlockSpec((BM, 1), lambda i: (i, 0)),
            pl.BlockSpec((8, 128), lambda i: (0, 0)),
            pl.BlockSpec((2, 8, 128), lambda i: (0, 0, 0)),
            pl.BlockSpec((8, 128), lambda i: (0, 0)),
            pl.BlockSpec((8, 128), lambda i: (0, 0)),
            pl.BlockSpec((1024, 512), lambda i: (0, 0)),
        ],
        out_specs=pl.BlockSpec((4, BM, 128), lambda i: (0, i, 0)),
        out_shape=jax.ShapeDtypeStruct((4, M, 128), F32),
    )(acc2, dis, b2, st2, g2, be2, W3)


def _tc_final(acc3, dis, b3, Wf_p, bf_p):
    BM = 1024

    def body(acc_ref, dis_ref, b_ref, w_ref, bf_ref, out_ref):
        d = dis_ref[...]
        fs = []
        for c in range(4):
            y = (acc_ref[0, c] + acc_ref[1, c]) * d + b_ref[c]
            fs.append(jnp.maximum(y, 0.0))
        f = jnp.concatenate(fs, axis=1)
        out_ref[...] = jnp.dot(f, w_ref[...],
                               preferred_element_type=F32) + bf_ref[...]

    return pl.pallas_call(
        body,
        grid=(M // BM,),
        in_specs=[
            pl.BlockSpec((2, 4, BM, 128), lambda i: (0, 0, i, 0)),
            pl.BlockSpec((BM, 1), lambda i: (i, 0)),
            pl.BlockSpec((4, 128), lambda i: (0, 0)),
            pl.BlockSpec((512, 128), lambda i: (0, 0)),
            pl.BlockSpec((1, 128), lambda i: (0, 0)),
        ],
        out_specs=pl.BlockSpec((BM, 128), lambda i: (i, 0)),
        out_shape=jax.ShapeDtypeStruct((M, 128), F32),
    )(acc3, dis, b3, Wf_p, bf_p)


def kernel(x, edge_index, edge_weight, W1, b1, g1, be1, W2, b2, g2, be2,
           W3, b3, Wf, bf):
    src = edge_index[0].astype(I32)
    dst = edge_index[1].astype(I32)
    loop = jnp.arange(N, dtype=I32)
    pad = EP - (E + N)
    src_w = jnp.concatenate([src, loop, jnp.zeros((pad,), I32)])
    dst_w = jnp.concatenate([dst, loop, jnp.zeros((pad,), I32)])
    w_w = jnp.concatenate([edge_weight.astype(F32), jnp.ones((N,), F32),
                           jnp.zeros((pad,), F32)])
    src_w, dst_w, w_w = jax.lax.sort((src_w, dst_w, w_w), num_keys=1)
    src_w = src_w.reshape(NW, NCHUNK, K)
    dst_w = dst_w.reshape(NW, NCHUNK, K)
    w_w = w_w.reshape(NW, NCHUNK, K)
    zeros2d = jnp.zeros((M, 128), F32)
    x_p = jnp.pad(x, ((0, M - N), (0, 0)))

    degp = _sc_degree(dst_w, w_w)
    dis, Xs = _tc_dis_scale(degp, x_p)

    acc0 = _sc_spmm(Xs, src_w, dst_w, w_w, zeros2d, 1).reshape(NC, M, 128)
    Y1, st1 = _tc_mm1_stats(acc0, dis, W1, b1.reshape(1, 1024))

    Z2 = _tc_bn_mm2(Y1, st1, g1.reshape(1, 1024), be1.reshape(1, 1024),
                    W2, dis)
    acc2 = _sc_spmm(Z2.reshape(8 * M, 128), src_w, dst_w, w_w, zeros2d, 8)
    st2 = _tc_stats2(acc2, dis, b2.reshape(8, 128))
    Z3 = _tc_bn_mm3(acc2, dis, b2.reshape(8, 128), st2, g2.reshape(8, 128),
                    be2.reshape(8, 128), W3)

    acc3 = _sc_spmm(Z3.reshape(4 * M, 128), src_w, dst_w, w_w, zeros2d, 4)
    Wf_p = jnp.pad(Wf, ((0, 0), (0, 118)))
    bf_p = jnp.pad(bf, (0, 118)).reshape(1, 128)
    O = _tc_final(acc3, dis, b3.reshape(4, 128), Wf_p, bf_p)
    return O[:N, :10]


# speedup vs baseline: 1.3992x; 1.3992x over previous
import functools

import jax
import jax.numpy as jnp
from jax import lax
from jax.experimental import pallas as pl
from jax.experimental.pallas import tpu as pltpu
from jax.experimental.pallas import tpu_sc as plsc

N = 10000
M = 10240
E = 320000
NC, NS, NW = 2, 16, 32
K = 128
EPW = 10496
NCHUNK = EPW // K
EP = NW * EPW
RPT = 624
F32 = jnp.float32
I32 = jnp.int32

_MESH = dict(core_axis_name="c", subcore_axis_name="s",
             num_cores=NC, num_subcores=NS)


def _sc_degree(dst_w, w_w):
    @functools.partial(
        pl.kernel,
        out_type=jax.ShapeDtypeStruct((NW, M), F32),
        mesh=plsc.VectorSubcoreMesh(**_MESH),
        compiler_params=pltpu.CompilerParams(needs_layout_passes=False),
        scratch_types=[
            pltpu.VMEM((NCHUNK, K), I32),
            pltpu.VMEM((NCHUNK, K), F32),
            pltpu.VMEM((M,), F32),
        ],
    )
    def deg_kernel(dst_hbm, w_hbm, degp_hbm, dst_v, w_v, acc_t):
        wid = lax.axis_index("c") * NS + lax.axis_index("s")
        pltpu.sync_copy(dst_hbm.at[wid], dst_v)
        pltpu.sync_copy(w_hbm.at[wid], w_v)

        def zbody(j, _):
            acc_t[pl.ds(j * 16, 16)] = jnp.zeros((16,), F32)
            return 0
        lax.fori_loop(0, M // 16, zbody, 0)

        def kbody(k, _):
            for j in range(K // 16):
                sl = pl.ds(j * 16, 16)
                plsc.addupdate_scatter(acc_t, [dst_v[k, sl]], w_v[k, sl])
            return 0
        lax.fori_loop(0, NCHUNK, kbody, 0)
        pltpu.sync_copy(acc_t, degp_hbm.at[wid])

    return deg_kernel(dst_w, w_w)


def _sc_spmm(table, src_w, dst_w, w_w, zeros2d, C):
    @functools.partial(
        pl.kernel,
        out_type=jax.ShapeDtypeStruct((NC, C, M, 128), F32),
        mesh=plsc.VectorSubcoreMesh(**_MESH),
        compiler_params=pltpu.CompilerParams(needs_layout_passes=False),
        scratch_types=[
            pltpu.VMEM((K,), I32), pltpu.VMEM((K,), I32),
            pltpu.VMEM((K,), I32), pltpu.VMEM((K,), I32),
            pltpu.VMEM((K,), F32), pltpu.VMEM((K,), F32),
            pltpu.VMEM((K,), I32), pltpu.VMEM((K,), I32),
            pltpu.VMEM((K,), I32), pltpu.VMEM((K,), I32),
            pltpu.VMEM((K, 128), F32), pltpu.VMEM((K, 128), F32),
            pltpu.VMEM_SHARED((N, 128), F32),
            pltpu.SemaphoreType.DMA, pltpu.SemaphoreType.DMA,
            pltpu.SemaphoreType.DMA, pltpu.SemaphoreType.DMA,
            pltpu.SemaphoreType.DMA, pltpu.SemaphoreType.DMA,
        ],
    )
    def spmm_kernel(table_hbm, src_hbm, dst_hbm, w_hbm, z_hbm, out_hbm,
                    src0, src1, dst0, dst1, w0, w1, idx0, idx1,
                    sdst0, sdst1, rows0, rows1, acc,
                    sg0, sg1, se0, se1, ss0, ss1):
        cid = lax.axis_index("c")
        sid = lax.axis_index("s")
        wid = cid * NS + sid
        rsl = pl.ds(sid * RPT, RPT)
        tail = pl.ds(NS * RPT, N - NS * RPT)
        SRC, DST, W = (src0, src1), (dst0, dst1), (w0, w1)
        IDX, ROWS = (idx0, idx1), (rows0, rows1)
        SDST = (sdst0, sdst1)
        SG, SE, SS = (sg0, sg1), (se0, se1), (ss0, ss1)
        NH = NCHUNK // 2

        def fetch_start(k, p):
            pltpu.async_copy(src_hbm.at[wid, k], SRC[p], SE[p])
            pltpu.async_copy(dst_hbm.at[wid, k], DST[p], SE[p])
            pltpu.async_copy(w_hbm.at[wid, k], W[p], SE[p])

        def fetch_wait(k, p):
            pltpu.make_async_copy(src_hbm.at[wid, k], SRC[p], SE[p]).wait()
            pltpu.make_async_copy(dst_hbm.at[wid, k], DST[p], SE[p]).wait()
            pltpu.make_async_copy(w_hbm.at[wid, k], W[p], SE[p]).wait()

        def idx_compute(p, c):
            for j in range(K // 16):
                sl = pl.ds(j * 16, 16)
                IDX[p][sl] = SRC[p][sl] + (c * M)
                SDST[p][sl] = DST[p][sl]

        def scale(p):
            def ebody(e, _):
                e16 = jnp.full((16,), e, I32)
                wv = plsc.load_gather(W[p], [e16])
                for j in range(8):
                    sl = pl.ds(j * 16, 16)
                    ROWS[p][e, sl] = ROWS[p][e, sl] * wv
                return 0
            lax.fori_loop(0, K, ebody, 0, unroll=8)

        for c in range(C):
            pltpu.sync_copy(z_hbm.at[rsl], acc.at[rsl])

            @pl.when(sid == 0)
            def _():
                pltpu.sync_copy(z_hbm.at[tail], acc.at[tail])

            plsc.subcore_barrier()

            pltpu.sync_copy(src_hbm.at[wid, 0], SRC[0])
            pltpu.sync_copy(dst_hbm.at[wid, 0], DST[0])
            pltpu.sync_copy(w_hbm.at[wid, 0], W[0])
            idx_compute(0, c)
            pltpu.async_copy(table_hbm.at[IDX[0]], ROWS[0], SG[0])
            fetch_start(1, 1)

            def kbody(i, _, c=c):
                for p in (0, 1):
                    q = 1 - p
                    k = 2 * i + p
                    pltpu.make_async_copy(
                        table_hbm.at[IDX[p]], ROWS[p], SG[p]).wait()

                    def prev_scatter_wait():
                        pltpu.make_async_copy(
                            ROWS[q], acc.at[SDST[q]], SS[q]).wait()

                    def next_gather():
                        fetch_wait(k + 1, q)
                        idx_compute(q, c)
                        h = K // 2
                        pltpu.async_copy(table_hbm.at[IDX[q].at[pl.ds(0, h)]],
                                         ROWS[q].at[pl.ds(0, h)], SG[q])
                        pltpu.async_copy(table_hbm.at[IDX[q].at[pl.ds(h, h)]],
                                         ROWS[q].at[pl.ds(h, h)], SG[q])

                    if p == 0:
                        @pl.when(i > 0)
                        def _():
                            prev_scatter_wait()
                        next_gather()
                    else:
                        prev_scatter_wait()

                        @pl.when(i < NH - 1)
                        def _():
                            next_gather()

                    scale(p)
                    pltpu.async_copy(ROWS[p], acc.at[SDST[p]], SS[p],
                                     add=True)

                    @pl.when(i < NH - 1)
                    def _():
                        fetch_start(k + 2, p)
                return 0

            lax.fori_loop(0, NH, kbody, 0)
            pltpu.make_async_copy(ROWS[1], acc.at[SDST[1]], SS[1]).wait()
            plsc.subcore_barrier()
            pltpu.sync_copy(acc.at[rsl], out_hbm.at[cid, c, rsl])

            @pl.when(sid == 0)
            def _():
                pltpu.sync_copy(acc.at[tail], out_hbm.at[cid, c, tail])

            plsc.subcore_barrier()

    return spmm_kernel(table, src_w, dst_w, w_w, zeros2d)


def _tc_dis_scale(degp, x_p):
    BM = 2048

    def body(degp_ref, x_ref, dis_ref, xs_ref):
        deg = jnp.sum(degp_ref[...], axis=0)
        dis = jnp.where(deg > 0, lax.rsqrt(deg), 0.0)[:, None]
        dis_ref[...] = dis
        xs_ref[...] = x_ref[...] * dis

    return pl.pallas_call(
        body,
        grid=(M // BM,),
        in_specs=[
            pl.BlockSpec((NW, BM), lambda i: (0, i)),
            pl.BlockSpec((BM, 128), lambda i: (i, 0)),
        ],
        out_specs=[
            pl.BlockSpec((BM, 1), lambda i: (i, 0)),
            pl.BlockSpec((BM, 128), lambda i: (i, 0)),
        ],
        out_shape=[
            jax.ShapeDtypeStruct((M, 1), F32),
            jax.ShapeDtypeStruct((M, 128), F32),
        ],
    )(degp, x_p)


def _tc_mm1_stats(acc0, dis, W1, b1):
    BM = 1024

    def body(acc_ref, dis_ref, w_ref, b_ref, y_ref, st_ref):
        i = pl.program_id(0)
        lhs = (acc_ref[0] + acc_ref[1]) * dis_ref[...]
        y = jnp.dot(lhs, w_ref[...], preferred_element_type=F32) + b_ref[...]
        y_ref[...] = y
        rows = i * BM + lax.broadcasted_iota(I32, (BM, 1), 0)
        ym = jnp.where(rows < N, y, 0.0)
        st = jnp.concatenate([jnp.sum(ym, axis=0, keepdims=True),
                              jnp.sum(ym * ym, axis=0, keepdims=True)], axis=0)

        @pl.when(i == 0)
        def _():
            st_ref[...] = st

        @pl.when(i > 0)
        def _():
            st_ref[...] = st_ref[...] + st

    return pl.pallas_call(
        body,
        grid=(M // BM,),
        in_specs=[
            pl.BlockSpec((2, BM, 128), lambda i: (0, i, 0)),
            pl.BlockSpec((BM, 1), lambda i: (i, 0)),
            pl.BlockSpec((128, 1024), lambda i: (0, 0)),
            pl.BlockSpec((1, 1024), lambda i: (0, 0)),
        ],
        out_specs=[
            pl.BlockSpec((BM, 1024), lambda i: (i, 0)),
            pl.BlockSpec((2, 1024), lambda i: (0, 0)),
        ],
        out_shape=[
            jax.ShapeDtypeStruct((M, 1024), F32),
            jax.ShapeDtypeStruct((2, 1024), F32),
        ],
    )(acc0, dis, W1, b1)


def _tc_bn_mm2(Y1, st1, g1, be1, W2, dis):
    BM = 512

    def body(y_ref, st_ref, g_ref, be_ref, w_ref, dis_ref, out_ref):
        m = st_ref[0] * (1.0 / N)
        v = st_ref[1] * (1.0 / N) - m * m
        scale = lax.rsqrt(v + 1e-5) * g_ref[0]
        shift = be_ref[0] - m * scale
        f = jnp.maximum(y_ref[...] * scale[None, :] + shift[None, :], 0.0)
        z = jnp.dot(f, w_ref[...], preferred_element_type=F32) * dis_ref[...]
        for c in range(8):
            out_ref[c] = z[:, c * 128:(c + 1) * 128]

    return pl.pallas_call(
        body,
        grid=(M // BM,),
        in_specs=[
            pl.BlockSpec((BM, 1024), lambda i: (i, 0)),
            pl.BlockSpec((2, 1024), lambda i: (0, 0)),
            pl.BlockSpec((1, 1024), lambda i: (0, 0)),
            pl.BlockSpec((1, 1024), lambda i: (0, 0)),
            pl.BlockSpec((1024, 1024), lambda i: (0, 0)),
            pl.BlockSpec((BM, 1), lambda i: (i, 0)),
        ],
        out_specs=pl.BlockSpec((8, BM, 128), lambda i: (0, i, 0)),
        out_shape=jax.ShapeDtypeStruct((8, M, 128), F32),
    )(Y1, st1, g1, be1, W2, dis)


def _tc_stats2(acc2, dis, b2):
    BM = 512

    def body(acc_ref, dis_ref, b_ref, st_ref):
        i = pl.program_id(0)
        rows = i * BM + lax.broadcasted_iota(I32, (BM, 1), 0)
        mask = rows < N
        ss_list, s_list = [], []
        for c in range(8):
            y = (acc_ref[0, c] + acc_ref[1, c]) * dis_ref[...] + b_ref[c]
            ym = jnp.where(mask, y, 0.0)
            s_list.append(jnp.sum(ym, axis=0))
            ss_list.append(jnp.sum(ym * ym, axis=0))
        st = jnp.stack([jnp.stack(s_list), jnp.stack(ss_list)])

        @pl.when(i == 0)
        def _():
            st_ref[...] = st

        @pl.when(i > 0)
        def _():
            st_ref[...] = st_ref[...] + st

    return pl.pallas_call(
        body,
        grid=(M // BM,),
        in_specs=[
            pl.BlockSpec((2, 8, BM, 128), lambda i: (0, 0, i, 0)),
            pl.BlockSpec((BM, 1), lambda i: (i, 0)),
            pl.BlockSpec((8, 128), lambda i: (0, 0)),
        ],
        out_specs=pl.BlockSpec((2, 8, 128), lambda i: (0, 0, 0)),
        out_shape=jax.ShapeDtypeStruct((2, 8, 128), F32),
    )(acc2, dis, b2)


def _tc_bn_mm3(acc2, dis, b2, st2, g2, be2, W3):
    BM = 512

    def body(acc_ref, dis_ref, b_ref, st_ref, g_ref, be_ref, w_ref, out_ref):
        d = dis_ref[...]
        fs = []
        for c in range(8):
            m = st_ref[0, c] * (1.0 / N)
            v = st_ref[1, c] * (1.0 / N) - m * m
            scale = lax.rsqrt(v + 1e-5) * g_ref[c]
            shift = be_ref[c] - m * scale
            y = (acc_ref[0, c] + acc_ref[1, c]) * d + b_ref[c]
            fs.append(jnp.maximum(y * scale[None, :] + shift[None, :], 0.0))
        f = jnp.concatenate(fs, axis=1)
        z = jnp.dot(f, w_ref[...], preferred_element_type=F32) * d
        for c in range(4):
            out_ref[c] = z[:, c * 128:(c + 1) * 128]

    return pl.pallas_call(
        body,
        grid=(M // BM,),
        in_specs=[
            pl.BlockSpec((2, 8, BM, 128), lambda i: (0, 0, i, 0)),
            pl.BlockSpec((BM, 1), lambda i: (i, 0)),
            pl.BlockSpec((8, 128), lambda i: (0, 0)),
            pl.BlockSpec((2, 8, 128), lambda i: (0, 0, 0)),
            pl.BlockSpec((8, 128), lambda i: (0, 0)),
            pl.BlockSpec((8, 128), lambda i: (0, 0)),
            pl.BlockSpec((1024, 512), lambda i: (0, 0)),
        ],
        out_specs=pl.BlockSpec((4, BM, 128), lambda i: (0, i, 0)),
        out_shape=jax.ShapeDtypeStruct((4, M, 128), F32),
    )(acc2, dis, b2, st2, g2, be2, W3)


def _tc_final(acc3, dis, b3, Wf_p, bf_p):
    BM = 1024

    def body(acc_ref, dis_ref, b_ref, w_ref, bf_ref, out_ref):
        d = dis_ref[...]
        fs = []
        for c in range(4):
            y = (acc_ref[0, c] + acc_ref[1, c]) * d + b_ref[c]
            fs.append(jnp.maximum(y, 0.0))
        f = jnp.concatenate(fs, axis=1)
        out_ref[...] = jnp.dot(f, w_ref[...],
                               preferred_element_type=F32) + bf_ref[...]

    return pl.pallas_call(
        body,
        grid=(M // BM,),
        in_specs=[
            pl.BlockSpec((2, 4, BM, 128), lambda i: (0, 0, i, 0)),
            pl.BlockSpec((BM, 1), lambda i: (i, 0)),
            pl.BlockSpec((4, 128), lambda i: (0, 0)),
            pl.BlockSpec((512, 128), lambda i: (0, 0)),
            pl.BlockSpec((1, 128), lambda i: (0, 0)),
        ],
        out_specs=pl.BlockSpec((BM, 128), lambda i: (i, 0)),
        out_shape=jax.ShapeDtypeStruct((M, 128), F32),
    )(acc3, dis, b3, Wf_p, bf_p)


def kernel(x, edge_index, edge_weight, W1, b1, g1, be1, W2, b2, g2, be2,
           W3, b3, Wf, bf):
    src = edge_index[0].astype(I32)
    dst = edge_index[1].astype(I32)
    loop = jnp.arange(N, dtype=I32)
    pad = EP - (E + N)
    src_w = jnp.concatenate([src, loop, jnp.zeros((pad,), I32)])
    dst_w = jnp.concatenate([dst, loop, jnp.zeros((pad,), I32)])
    w_w = jnp.concatenate([edge_weight.astype(F32), jnp.ones((N,), F32),
                           jnp.zeros((pad,), F32)])
    src_w = src_w.reshape(NW, NCHUNK, K)
    dst_w = dst_w.reshape(NW, NCHUNK, K)
    w_w = w_w.reshape(NW, NCHUNK, K)
    zeros2d = jnp.zeros((M, 128), F32)
    x_p = jnp.pad(x, ((0, M - N), (0, 0)))

    degp = _sc_degree(dst_w, w_w)
    dis, Xs = _tc_dis_scale(degp, x_p)

    acc0 = _sc_spmm(Xs, src_w, dst_w, w_w, zeros2d, 1).reshape(NC, M, 128)
    Y1, st1 = _tc_mm1_stats(acc0, dis, W1, b1.reshape(1, 1024))

    Z2 = _tc_bn_mm2(Y1, st1, g1.reshape(1, 1024), be1.reshape(1, 1024),
                    W2, dis)
    acc2 = _sc_spmm(Z2.reshape(8 * M, 128), src_w, dst_w, w_w, zeros2d, 8)
    st2 = _tc_stats2(acc2, dis, b2.reshape(8, 128))
    Z3 = _tc_bn_mm3(acc2, dis, b2.reshape(8, 128), st2, g2.reshape(8, 128),
                    be2.reshape(8, 128), W3)

    acc3 = _sc_spmm(Z3.reshape(4 * M, 128), src_w, dst_w, w_w, zeros2d, 4)
    Wf_p = jnp.pad(Wf, ((0, 0), (0, 118)))
    bf_p = jnp.pad(bf, (0, 118)).reshape(1, 128)
    O = _tc_final(acc3, dis, b3.reshape(4, 128), Wf_p, bf_p)
    return O[:N, :10]
